# baseline (speedup 1.0000x reference)
import jax
import jax.numpy as jnp
from jax.experimental import pallas as pl
from jax.experimental.pallas import tpu as pltpu
from jax.experimental.pallas import tpu_sc as plsc

NUM_ROWS = 1000000
BATCH = 16384
DIM = 64
WINDOW = 128
LANES = 16


def kernel(user, item, user_table, item_table):
    utF = user_table.T.reshape(DIM * NUM_ROWS)
    itF = item_table.T.reshape(DIM * NUM_ROWS)
    u2 = user.reshape(1, BATCH)
    i2 = item.reshape(1, BATCH)

    mesh = plsc.VectorSubcoreMesh(core_axis_name="core",
                                  subcore_axis_name="subcore")

    @pl.kernel(
        out_type=jax.ShapeDtypeStruct((DIM, BATCH), jnp.float32),
        mesh=mesh,
        compiler_params=pltpu.CompilerParams(use_tc_tiling_on_sc=False),
        scratch_types=[
            pltpu.VMEM((DIM * WINDOW,), jnp.float32),
            pltpu.VMEM((DIM * WINDOW,), jnp.float32),
            pltpu.VMEM((DIM * WINDOW,), jnp.int32),
            pltpu.VMEM((DIM * WINDOW,), jnp.int32),
            pltpu.SemaphoreType.DMA,
            pltpu.SemaphoreType.DMA,
        ],
    )
    def sc_kernel(u_hbm, i_hbm, ut_hbm, it_hbm, o_hbm,
                  ubuf, ibuf, uidx, iidx, sem_u, sem_i):
        def body(u_idx, i_idx, o_vmem):
            @pl.loop(0, DIM)
            def _(d):
                @pl.loop(0, WINDOW, step=LANES)
                def _(c):
                    src = (pl.ds(0, 1), pl.ds(c, LANES))
                    uidx.at[pl.ds(d * WINDOW + c, LANES)][...] = (
                        u_idx.at[*src][...].reshape(LANES) + d * NUM_ROWS)
                    iidx.at[pl.ds(d * WINDOW + c, LANES)][...] = (
                        i_idx.at[*src][...].reshape(LANES) + d * NUM_ROWS)

            cp_u = pltpu.make_async_copy(ut_hbm.at[uidx], ubuf, sem_u)
            cp_i = pltpu.make_async_copy(it_hbm.at[iidx], ibuf, sem_i)
            cp_u.start()
            cp_i.start()
            cp_u.wait()
            cp_i.wait()

            @pl.loop(0, DIM)
            def _(d):
                @pl.loop(0, WINDOW, step=LANES)
                def _(c):
                    flat = pl.ds(d * WINDOW + c, LANES)
                    o_vmem.at[pl.ds(d, 1), pl.ds(c, LANES)][...] = (
                        ubuf.at[flat][...] * ibuf.at[flat][...]
                    ).reshape(1, LANES)

        pltpu.emit_pipeline(
            body,
            grid=(BATCH // WINDOW,),
            in_specs=[
                pl.BlockSpec((1, WINDOW), lambda i: (0, i)),
                pl.BlockSpec((1, WINDOW), lambda i: (0, i)),
            ],
            out_specs=[pl.BlockSpec((DIM, WINDOW), lambda i: (0, i))],
            core_axis_name=("core", "subcore"),
            dimension_semantics=(pltpu.PARALLEL,),
        )(u_hbm, i_hbm, o_hbm)

    out = sc_kernel(u2, i2, utF, itF)
    return out.T
